# Initial kernel scaffold; baseline (speedup 1.0000x reference)
import jax
import jax.numpy as jnp
from jax.experimental import pallas as pl


def kernel(input, rois):
    raise NotImplementedError("write your pallas kernel here")


# speedup vs baseline: 9.8449x; 9.8449x over previous
import functools
import math

import jax
import jax.numpy as jnp
from jax import lax
from jax.experimental import pallas as pl
from jax.experimental.pallas import tpu as pltpu
from jax.experimental.pallas import tpu_sc as plsc

OUT_HW = 7
SAMPLES = 2
SCALE = 0.25
NW = 32
LANES = 16


def _make_sc_kernel(R, n_per, num_rows, Hp, Wp, C):
    CCH = C // LANES
    OUTROW = OUT_HW * OUT_HW * C
    mesh = plsc.VectorSubcoreMesh(core_axis_name="c", subcore_axis_name="s")

    @functools.partial(
        pl.kernel,
        mesh=mesh,
        compiler_params=pltpu.CompilerParams(needs_layout_passes=False),
        out_type=jax.ShapeDtypeStruct((R, OUTROW), jnp.float32),
        scratch_types=[
            pltpu.VMEM((n_per * 16,), jnp.float32),
            pltpu.VMEM((32,), jnp.int32),
            pltpu.VMEM((32,), jnp.int32),
            pltpu.VMEM((32,), jnp.float32),
            pltpu.VMEM((32,), jnp.float32),
            pltpu.VMEM((32,), jnp.float32),
            pltpu.VMEM((16, C), jnp.float32),
            pltpu.VMEM((OUTROW,), jnp.float32),
            pltpu.SemaphoreType.DMA,
        ],
    )
    def k(table_h, rois_h, out_h, rois_v, ycomb, xcomb, wyv, wxv, wv,
          patch, roibuf, sem):
        cid = lax.axis_index("c")
        sid = lax.axis_index("s")
        wid = sid * 2 + cid
        start = wid * n_per
        cnt = jnp.maximum(0, jnp.minimum(n_per, R - start))
        pltpu.sync_copy(rois_h.at[pl.ds(start * 16, n_per * 16)], rois_v)

        lane = lax.broadcasted_iota(jnp.int32, (LANES,), 0)
        offs = ((lane >> 1).astype(jnp.float32)
                + 0.25 + 0.5 * (lane & 1).astype(jnp.float32))
        gy_base = 16 * ((lane >> 1) & 1) + ((lane >> 3) & 1)
        gx_base = 16 * (lane & 1) + ((lane >> 2) & 1)

        def roi_body(kk, carry):
            r = start + kk

            def bc(j):
                jv = jnp.broadcast_to(kk * 16 + j, (LANES,))
                return plsc.load_gather(rois_v, [jv])

            b_f = bc(0)
            x1 = bc(1) * SCALE - 0.5
            y1 = bc(2) * SCALE - 0.5
            x2 = bc(3) * SCALE - 0.5
            y2 = bc(4) * SCALE - 0.5
            bh = (y2 - y1) * (1.0 / OUT_HW)
            bw = (x2 - x1) * (1.0 / OUT_HW)
            ys = y1 + bh * offs
            xs = x1 + bw * offs
            vy = jnp.where((ys >= -1.0) & (ys <= jnp.float32(Hp - 1)),
                           jnp.float32(0.5), jnp.float32(0.0))
            vx = jnp.where((xs >= -1.0) & (xs <= jnp.float32(Wp - 1)),
                           jnp.float32(0.5), jnp.float32(0.0))
            yc = jnp.clip(ys, 0.0, jnp.float32(Hp - 2))
            xc = jnp.clip(xs, 0.0, jnp.float32(Wp - 2))
            yl = yc.astype(jnp.int32)
            xl = xc.astype(jnp.int32)
            ly = yc - yl.astype(jnp.float32)
            lx = xc - xl.astype(jnp.float32)
            row0 = (b_f.astype(jnp.int32) * Hp + yl) * Wp
            ycomb[pl.ds(0, 16)] = row0
            ycomb[pl.ds(16, 16)] = row0 + Wp
            xcomb[pl.ds(0, 16)] = xl
            xcomb[pl.ds(16, 16)] = xl + 1
            wyv[pl.ds(0, 16)] = (1.0 - ly) * vy
            wyv[pl.ds(16, 16)] = ly * vy
            wxv[pl.ds(0, 16)] = (1.0 - lx) * vx
            wxv[pl.ds(16, 16)] = lx * vx

            def bin_body(bi, c2):
                py = bi // OUT_HW
                px = bi - py * OUT_HW
                gy = gy_base + 2 * py
                gx = gx_base + 2 * px
                idxv = (plsc.load_gather(ycomb, [gy])
                        + plsc.load_gather(xcomb, [gx]))
                wv[pl.ds(16, 16)] = (plsc.load_gather(wyv, [gy])
                                     * plsc.load_gather(wxv, [gx]))
                pltpu.async_copy(table_h.at[idxv], patch, sem).wait()
                accs = [jnp.zeros((LANES,), jnp.float32) for _ in range(CCH)]
                for t in range(16):
                    wb = plsc.load_gather(
                        wv, [jnp.broadcast_to(jnp.int32(16 + t), (LANES,))])
                    for c in range(CCH):
                        accs[c] = accs[c] + wb * patch[t, pl.ds(c * LANES, LANES)]
                boff = bi * C
                for c in range(CCH):
                    roibuf[pl.ds(boff + c * LANES, LANES)] = accs[c]
                return c2

            lax.fori_loop(0, OUT_HW * OUT_HW, bin_body, 0)
            pltpu.sync_copy(roibuf, out_h.at[r])
            return carry

        lax.fori_loop(0, cnt, roi_body, 0)

    return k


def kernel(input, rois):
    N, C, H, W = input.shape
    R = rois.shape[0]
    Hp, Wp = H + 1, W + 1
    feat = jnp.pad(jnp.transpose(input, (0, 2, 3, 1)),
                   ((0, 0), (0, 1), (0, 1), (0, 0)))
    table = feat.reshape(N * Hp * Wp, C)
    n_per = 8 * int(math.ceil(R / (8 * NW)))
    rois_p = jnp.pad(rois, ((0, NW * n_per - R), (0, 16 - rois.shape[1])))
    rois_p = rois_p.reshape(-1)
    out = _make_sc_kernel(R, n_per, N * Hp * Wp, Hp, Wp, C)(table, rois_p)
    return out.reshape(R, OUT_HW, OUT_HW, C).transpose(0, 3, 1, 2)


# speedup vs baseline: 10.6201x; 1.0787x over previous
import functools
import math

import jax
import jax.numpy as jnp
from jax import lax
from jax.experimental import pallas as pl
from jax.experimental.pallas import tpu as pltpu
from jax.experimental.pallas import tpu_sc as plsc

OUT_HW = 7
SAMPLES = 2
SCALE = 0.25
NW = 32
LANES = 16


def _make_sc_kernel(R, n_per, num_rows, Hp, Wp, C):
    CCH = C // LANES
    OUTROW = OUT_HW * OUT_HW * C
    mesh = plsc.VectorSubcoreMesh(core_axis_name="c", subcore_axis_name="s")

    @functools.partial(
        pl.kernel,
        mesh=mesh,
        compiler_params=pltpu.CompilerParams(needs_layout_passes=False),
        out_type=jax.ShapeDtypeStruct((R, OUTROW), jnp.float32),
        scratch_types=[
            pltpu.VMEM((n_per * 16,), jnp.float32),
            pltpu.VMEM((32,), jnp.int32),
            pltpu.VMEM((32,), jnp.int32),
            pltpu.VMEM((32,), jnp.float32),
            pltpu.VMEM((32,), jnp.float32),
            pltpu.VMEM((OUT_HW * 16,), jnp.int32),
            pltpu.VMEM((128,), jnp.float32),
            pltpu.VMEM((OUT_HW * 16, C), jnp.float32),
            pltpu.VMEM((OUTROW,), jnp.float32),
            pltpu.SemaphoreType.DMA,
        ],
    )
    def k(table_h, rois_h, out_h, rois_v, ycomb, xcomb, wyv, wxv, idxbuf,
          wvbuf, patch, roibuf, sem):
        cid = lax.axis_index("c")
        sid = lax.axis_index("s")
        wid = sid * 2 + cid
        start = wid * n_per
        cnt = jnp.maximum(0, jnp.minimum(n_per, R - start))
        pltpu.sync_copy(rois_h.at[pl.ds(start * 16, n_per * 16)], rois_v)

        lane = lax.broadcasted_iota(jnp.int32, (LANES,), 0)
        offs = ((lane >> 1).astype(jnp.float32)
                + 0.25 + 0.5 * (lane & 1).astype(jnp.float32))
        gy_base = 16 * ((lane >> 1) & 1) + ((lane >> 3) & 1)
        gx_base = 16 * (lane & 1) + ((lane >> 2) & 1)

        def roi_body(kk, carry):
            r = start + kk

            def bc(j):
                jv = jnp.broadcast_to(kk * 16 + j, (LANES,))
                return plsc.load_gather(rois_v, [jv])

            b_f = bc(0)
            x1 = bc(1) * SCALE - 0.5
            y1 = bc(2) * SCALE - 0.5
            x2 = bc(3) * SCALE - 0.5
            y2 = bc(4) * SCALE - 0.5
            bh = (y2 - y1) * (1.0 / OUT_HW)
            bw = (x2 - x1) * (1.0 / OUT_HW)
            ys = y1 + bh * offs
            xs = x1 + bw * offs
            vy = jnp.where((ys >= -1.0) & (ys <= jnp.float32(Hp - 1)),
                           jnp.float32(0.5), jnp.float32(0.0))
            vx = jnp.where((xs >= -1.0) & (xs <= jnp.float32(Wp - 1)),
                           jnp.float32(0.5), jnp.float32(0.0))
            yc = jnp.clip(ys, 0.0, jnp.float32(Hp - 2))
            xc = jnp.clip(xs, 0.0, jnp.float32(Wp - 2))
            yl = yc.astype(jnp.int32)
            xl = xc.astype(jnp.int32)
            ly = yc - yl.astype(jnp.float32)
            lx = xc - xl.astype(jnp.float32)
            row0 = (b_f.astype(jnp.int32) * Hp + yl) * Wp
            ycomb[pl.ds(0, 16)] = row0
            ycomb[pl.ds(16, 16)] = row0 + Wp
            xcomb[pl.ds(0, 16)] = xl
            xcomb[pl.ds(16, 16)] = xl + 1
            wyv[pl.ds(0, 16)] = (1.0 - ly) * vy
            wyv[pl.ds(16, 16)] = ly * vy
            wxv[pl.ds(0, 16)] = (1.0 - lx) * vx
            wxv[pl.ds(16, 16)] = lx * vx

            def row_body(ry, c2):
                gy = gy_base + 2 * ry
                rowy = plsc.load_gather(ycomb, [gy])
                wyrow = plsc.load_gather(wyv, [gy])
                for b in range(OUT_HW):
                    gx = gx_base + 2 * b
                    idxbuf[pl.ds(b * 16, 16)] = (
                        rowy + plsc.load_gather(xcomb, [gx]))
                    wvbuf[pl.ds(16 + b * 16, 16)] = (
                        wyrow * plsc.load_gather(wxv, [gx]))
                pltpu.async_copy(table_h.at[idxbuf], patch, sem).wait()
                for b in range(OUT_HW):
                    accs = [jnp.zeros((LANES,), jnp.float32)
                            for _ in range(CCH)]
                    for t in range(16):
                        wb = plsc.load_gather(
                            wvbuf,
                            [jnp.broadcast_to(jnp.int32(16 + b * 16 + t),
                                              (LANES,))])
                        for c in range(CCH):
                            accs[c] = accs[c] + wb * patch[
                                b * 16 + t, pl.ds(c * LANES, LANES)]
                    boff = (ry * OUT_HW + b) * C
                    for c in range(CCH):
                        roibuf[pl.ds(boff + c * LANES, LANES)] = accs[c]
                return c2

            lax.fori_loop(0, OUT_HW, row_body, 0)
            pltpu.sync_copy(roibuf, out_h.at[r])
            return carry

        lax.fori_loop(0, cnt, roi_body, 0)

    return k


def kernel(input, rois):
    N, C, H, W = input.shape
    R = rois.shape[0]
    Hp, Wp = H + 1, W + 1
    feat = jnp.pad(jnp.transpose(input, (0, 2, 3, 1)),
                   ((0, 0), (0, 1), (0, 1), (0, 0)))
    table = feat.reshape(N * Hp * Wp, C)
    n_per = 8 * int(math.ceil(R / (8 * NW)))
    rois_p = jnp.pad(rois, ((0, NW * n_per - R), (0, 16 - rois.shape[1])))
    rois_p = rois_p.reshape(-1)
    out = _make_sc_kernel(R, n_per, N * Hp * Wp, Hp, Wp, C)(table, rois_p)
    return out.reshape(R, OUT_HW, OUT_HW, C).transpose(0, 3, 1, 2)
